# baseline (speedup 1.0000x reference)
import functools

import jax
import jax.numpy as jnp
from jax import lax
from jax.experimental import pallas as pl
from jax.experimental.pallas import tpu as pltpu
from jax.experimental.pallas import tpu_sc as plsc

_NC = 2
_NS = 16
_NW = _NC * _NS

_D = 64
_DP = 128
_L = 16
_NA = 128
_NB = 72


def _make_lookup(S, T):
    assert T == _NA + _NB
    assert S % (2 * _NW) == 0
    s_per_w = S // _NW
    idx_per_w = s_per_w * T
    mesh = plsc.VectorSubcoreMesh(core_axis_name="c", subcore_axis_name="s")

    @functools.partial(
        pl.kernel,
        out_type=jax.ShapeDtypeStruct((S, T, _D), jnp.float32),
        mesh=mesh,
        scratch_types=[
            pltpu.VMEM((idx_per_w,), jnp.int32),
            pltpu.VMEM((_NA, _DP), jnp.float32),
            pltpu.VMEM((_NA, _DP), jnp.float32),
            pltpu.VMEM((_NB, _DP), jnp.float32),
            pltpu.VMEM((_NB, _DP), jnp.float32),
            pltpu.VMEM((_NA, _D), jnp.float32),
            pltpu.VMEM((_NB, _D), jnp.float32),
            pltpu.SemaphoreType.DMA,
            pltpu.SemaphoreType.DMA,
            pltpu.SemaphoreType.DMA,
            pltpu.SemaphoreType.DMA,
        ],
    )
    def lookup(table_hbm, idx_hbm, out_hbm, idx_v, ga0, ga1, gb0, gb1,
               pa, pb, sa0, sa1, sb0, sb1):
        wid = lax.axis_index("s") * _NC + lax.axis_index("c")
        s_base = wid * s_per_w
        pltpu.sync_copy(idx_hbm.at[wid], idx_v)

        gas = (ga0, ga1)
        gbs = (gb0, gb1)
        sas = (sa0, sa1)
        sbs = (sb0, sb1)

        def fire(s, b):
            off = pl.multiple_of(s * T, 8)
            pltpu.async_copy(
                table_hbm.at[idx_v.at[pl.ds(off, _NA)]], gas[b], sas[b]
            )
            pltpu.async_copy(
                table_hbm.at[idx_v.at[pl.ds(off + _NA, _NB)]], gbs[b], sbs[b]
            )

        def drain_a(b):
            pltpu.make_async_copy(
                table_hbm.at[idx_v.at[pl.ds(0, _NA)]], gas[b], sas[b]
            ).wait()

        def drain_b(b):
            pltpu.make_async_copy(
                table_hbm.at[idx_v.at[pl.ds(0, _NB)]], gbs[b], sbs[b]
            ).wait()

        def repack(src, dst, nrows):
            def blk(i, _):
                r0 = i * 8
                for j in range(8):
                    for k in range(_D // _L):
                        dst[r0 + j, pl.ds(k * _L, _L)] = src[
                            r0 + j, pl.ds(k * _L, _L)
                        ]
                return 0

            lax.fori_loop(0, nrows // 8, blk, 0)

        def process(s, b):
            sg = s_base + s
            drain_a(b)
            repack(gas[b], pa, _NA)
            pltpu.sync_copy(pa, out_hbm.at[sg, pl.ds(0, _NA)])
            drain_b(b)
            repack(gbs[b], pb, _NB)
            pltpu.sync_copy(pb, out_hbm.at[sg, pl.ds(_NA, _NB)])

        fire(0, 0)

        def body(i0, _):
            s0 = 2 * i0
            fire(s0 + 1, 1)
            process(s0, 0)
            fire(s0 + 2, 0)
            process(s0 + 1, 1)
            return 0

        lax.fori_loop(0, (s_per_w - 2) // 2, body, 0)

        fire(s_per_w - 1, 1)
        process(s_per_w - 2, 0)
        process(s_per_w - 1, 1)

    return lookup


def kernel(sen_ids, table):
    S, T = sen_ids.shape
    table_p = lax.pad(table, jnp.float32(0), ((0, 0, 0), (0, _DP - _D, 0)))
    idx2 = sen_ids.astype(jnp.int32).reshape(_NW, (S // _NW) * T)
    return _make_lookup(S, T)(table_p, idx2)


# speedup vs baseline: 1.1952x; 1.1952x over previous
import functools

import jax
import jax.numpy as jnp
from jax import lax
from jax.experimental import pallas as pl
from jax.experimental.pallas import tpu as pltpu
from jax.experimental.pallas import tpu_sc as plsc

_NC = 2
_NS = 16
_NW = _NC * _NS

_D = 64
_DP = 128
_CHUNK = 128
_NBUF = 4


def _make_lookup(B):
    assert B % (_NW * _CHUNK) == 0
    per_w = B // _NW
    nch = per_w // _CHUNK
    assert nch % _NBUF == 0
    mesh = plsc.VectorSubcoreMesh(core_axis_name="c", subcore_axis_name="s")

    @functools.partial(
        pl.kernel,
        out_type=jax.ShapeDtypeStruct((B, _DP), jnp.float32),
        mesh=mesh,
        scratch_types=[
            pltpu.VMEM((nch, _CHUNK), jnp.int32),
        ]
        + [pltpu.VMEM((_CHUNK, _DP), jnp.float32) for _ in range(_NBUF)]
        + [pltpu.SemaphoreType.DMA for _ in range(_NBUF)],
    )
    def lookup(table_hbm, idx_hbm, out_hbm, idx_v, *bufsems):
        bufs = bufsems[:_NBUF]
        sems = bufsems[_NBUF:]
        wid = lax.axis_index("s") * _NC + lax.axis_index("c")
        base = pl.multiple_of(wid * per_w, _CHUNK)
        pltpu.sync_copy(idx_hbm.at[wid], idx_v)

        def fire(t, b):
            pltpu.async_copy(table_hbm.at[idx_v.at[t]], bufs[b], sems[b])

        def drain(b):
            pltpu.make_async_copy(
                table_hbm.at[idx_v.at[0]], bufs[b], sems[b]
            ).wait()

        def copy_out(t, b):
            off = pl.multiple_of(base + t * _CHUNK, _CHUNK)
            pltpu.sync_copy(bufs[b], out_hbm.at[pl.ds(off, _CHUNK)])

        for b in range(_NBUF):
            fire(b, b)

        def body(i0, _):
            t0 = _NBUF * i0
            for b in range(_NBUF):
                drain(b)
                copy_out(t0 + b, b)
                fire(t0 + b + _NBUF, b)
            return 0

        lax.fori_loop(0, (nch - _NBUF) // _NBUF, body, 0)

        for b in range(_NBUF):
            drain(b)
            copy_out(nch - _NBUF + b, b)

    return lookup


def kernel(sen_ids, table):
    S, T = sen_ids.shape
    B = S * T
    table_p = lax.pad(table, jnp.float32(0), ((0, 0, 0), (0, _DP - _D, 0)))
    idx = sen_ids.reshape(-1).astype(jnp.int32)
    idx3 = idx.reshape(_NW, B // (_NW * _CHUNK), _CHUNK)
    out = _make_lookup(B)(table_p, idx3)
    return out.reshape(S, T, _DP)[..., :_D]
